# baseline (speedup 1.0000x reference)
import functools

import jax
import jax.numpy as jnp
from jax.experimental import pallas as pl
from jax.experimental.pallas import tpu as pltpu

_EPS = 1e-5
_B = 4
_C = 2


def _fold_bn(g, b, m, v):
    s = g / jnp.sqrt(v + _EPS)
    return s, b - m * s


def _block_body(x_ref, w1_ref, w2_ref, sb_ref, o_ref, h1_ref, xp_ref, *,
                H, W, Wp, Co):
    R = H * Wp
    F = (H + 2) * Wp
    Cin = x_ref.shape[2]

    s1 = sb_ref[0:1].reshape(1, 1, Co)
    b1 = sb_ref[1:2].reshape(1, 1, Co)
    s2 = sb_ref[2:3].reshape(1, 1, Co)
    b2 = sb_ref[3:4].reshape(1, 1, Co)
    ssc = sb_ref[4:5].reshape(1, 1, Co)
    bsc = sb_ref[5:6].reshape(1, 1, Co)

    xp_ref[...] = jnp.zeros((_B * (H + 2), Wp, Cin), jnp.bfloat16)
    for j in range(_B):
        xp_ref[j * (H + 2) + 1:j * (H + 2) + 1 + H, 1:W + 1, :] = (
            x_ref[j].reshape(H, W, Cin))
    h1_ref[...] = jnp.zeros((_B * (H + 2), Wp, Co), jnp.bfloat16)

    for g in range(_B // _C):
        xflat = xp_ref[g * _C * (H + 2):(g + 1) * _C * (H + 2)].reshape(
            _C * F, Cin)
        pieces = []
        for j in range(_C):
            xm = xflat[j * F:(j + 1) * F]
            pieces.append(jnp.concatenate(
                [xm[0:R], xm[Wp:Wp + R], xm[2 * Wp:2 * Wp + R]], axis=1))
        xcat = jnp.concatenate(pieces, axis=0)
        p1 = jnp.dot(xcat, w1_ref[...], preferred_element_type=jnp.float32)
        p1 = p1.reshape(_C * H, Wp, 4 * Co)
        acc1 = (p1[:, 0:W, 0:Co] + p1[:, 1:W + 1, Co:2 * Co]
                + p1[:, 2:W + 2, 2 * Co:3 * Co])
        sc = p1[:, 1:W + 1, 3 * Co:4 * Co]
        h1 = jnp.maximum(acc1 * s1 + b1, 0.0).astype(jnp.bfloat16)

        for j in range(_C):
            jj = g * _C + j
            h1_ref[jj * (H + 2) + 1:jj * (H + 2) + 1 + H, 1:W + 1, :] = (
                h1[j * H:(j + 1) * H])
        h1f = h1_ref[g * _C * (H + 2):(g + 1) * _C * (H + 2)].reshape(
            _C * F, Co)

        pieces2 = []
        for j in range(_C):
            hj = h1f[j * F:(j + 1) * F]
            pieces2.append(jnp.concatenate(
                [hj[0:R], hj[Wp:Wp + R], hj[2 * Wp:2 * Wp + R]], axis=1))
        xcat2 = jnp.concatenate(pieces2, axis=0)
        p2 = jnp.dot(xcat2, w2_ref[...], preferred_element_type=jnp.float32)
        p2 = p2.reshape(_C * H, Wp, 3 * Co)
        acc2 = (p2[:, 0:W, 0:Co] + p2[:, 1:W + 1, Co:2 * Co]
                + p2[:, 2:W + 2, 2 * Co:3 * Co])

        out = acc2 * s2 + b2 + (sc * ssc + bsc)
        out = jnp.maximum(out, 0.0)
        o_ref[g * _C:(g + 1) * _C] = out.reshape(_C, H, W, Co)


def kernel(x, w1, g1, b1, m1, v1, w2, g2, b2, m2, v2,
           w_sc, g_sc, b_sc, m_sc, v_sc):
    N, Cin, H, W = x.shape
    Co = w1.shape[0]
    Wp = 64
    Hp = H + 2

    xf = jnp.transpose(x, (0, 2, 3, 1)).astype(jnp.bfloat16)
    xf = xf.reshape(N, H * W, Cin)

    s1, bb1 = _fold_bn(g1, b1, m1, v1)
    s2, bb2 = _fold_bn(g2, b2, m2, v2)
    ssc, bbsc = _fold_bn(g_sc, b_sc, m_sc, v_sc)
    z = jnp.zeros_like(s1)
    sb = jnp.stack([s1, bb1, s2, bb2, ssc, bbsc, z, z])

    wt1 = jnp.transpose(w1, (2, 1, 3, 0)).reshape(3 * Cin, 3 * Co)
    scb = jnp.zeros((3 * Cin, Co), jnp.float32)
    scb = scb.at[Cin:2 * Cin].set(w_sc[:, :, 0, 0].T)
    w1cat = jnp.concatenate([wt1, scb], axis=1).astype(jnp.bfloat16)
    w2cat = jnp.transpose(w2, (2, 1, 3, 0)).reshape(3 * Co, 3 * Co)
    w2cat = w2cat.astype(jnp.bfloat16)

    body = functools.partial(_block_body, H=H, W=W, Wp=Wp, Co=Co)
    out = pl.pallas_call(
        body,
        out_shape=jax.ShapeDtypeStruct((N, H, W, Co), jnp.float32),
        grid=(N // _B,),
        in_specs=[
            pl.BlockSpec((_B, H * W, Cin), lambda i: (i, 0, 0)),
            pl.BlockSpec((3 * Cin, 4 * Co), lambda i: (0, 0)),
            pl.BlockSpec((3 * Co, 3 * Co), lambda i: (0, 0)),
            pl.BlockSpec((8, Co), lambda i: (0, 0)),
        ],
        out_specs=pl.BlockSpec((_B, H, W, Co), lambda i: (i, 0, 0, 0)),
        scratch_shapes=[pltpu.VMEM((_B * Hp, Wp, Co), jnp.bfloat16),
                        pltpu.VMEM((_B * Hp, Wp, Cin), jnp.bfloat16)],
        compiler_params=pltpu.CompilerParams(
            dimension_semantics=("parallel",)),
    )(xf, w1cat, w2cat, sb)
    return jnp.transpose(out, (0, 3, 1, 2))


# speedup vs baseline: 1.0832x; 1.0832x over previous
import functools

import jax
import jax.numpy as jnp
from jax.experimental import pallas as pl
from jax.experimental.pallas import tpu as pltpu

_EPS = 1e-5
_B = 2


def _fold_bn(g, b, m, v):
    s = g / jnp.sqrt(v + _EPS)
    return s, b - m * s


def _block_body(x_ref, w1_ref, w2_ref, sb_ref, o_ref, h1_ref, xp_ref, *,
                H, W, Wp, Co):
    R = H * Wp
    F = (H + 2) * Wp
    Cin = x_ref.shape[2]
    for j in range(_B):
        b0 = j * (H + 2)
        xp_ref[b0:b0 + 1] = jnp.zeros((1, Wp, Cin), jnp.bfloat16)
        xp_ref[b0 + H + 1:b0 + H + 2] = jnp.zeros((1, Wp, Cin), jnp.bfloat16)
        xp_ref[b0 + 1:b0 + 1 + H, 0:1, :] = jnp.zeros((H, 1, Cin),
                                                      jnp.bfloat16)
        xp_ref[b0 + 1:b0 + 1 + H, W + 1:Wp, :] = jnp.zeros(
            (H, Wp - W - 1, Cin), jnp.bfloat16)
        xp_ref[b0 + 1:b0 + 1 + H, 1:W + 1, :] = x_ref[j].reshape(H, W, Cin)
    xflat = xp_ref[...].reshape(_B * F, Cin)
    pieces = []
    for j in range(_B):
        xm = xflat[j * F:(j + 1) * F]
        pieces.append(jnp.concatenate(
            [xm[0:R], xm[Wp:Wp + R], xm[2 * Wp:2 * Wp + R]], axis=1))
    xcat = jnp.concatenate(pieces, axis=0)
    p1 = jnp.dot(xcat, w1_ref[...], preferred_element_type=jnp.float32)
    p1 = p1.reshape(_B * H, Wp, 4 * Co)
    acc1 = (p1[:, 0:W, 0:Co] + p1[:, 1:W + 1, Co:2 * Co]
            + p1[:, 2:W + 2, 2 * Co:3 * Co])
    sc = p1[:, 1:W + 1, 3 * Co:4 * Co]

    s1 = sb_ref[0:1].reshape(1, 1, Co)
    b1 = sb_ref[1:2].reshape(1, 1, Co)
    h1 = jnp.maximum(acc1 * s1 + b1, 0.0).astype(jnp.bfloat16)

    for j in range(_B):
        b0 = j * (H + 2)
        h1_ref[b0:b0 + 1] = jnp.zeros((1, Wp, Co), jnp.bfloat16)
        h1_ref[b0 + H + 1:b0 + H + 2] = jnp.zeros((1, Wp, Co), jnp.bfloat16)
        h1_ref[b0 + 1:b0 + 1 + H, 0:1, :] = jnp.zeros((H, 1, Co),
                                                      jnp.bfloat16)
        h1_ref[b0 + 1:b0 + 1 + H, W + 1:Wp, :] = jnp.zeros(
            (H, Wp - W - 1, Co), jnp.bfloat16)
        h1_ref[b0 + 1:b0 + 1 + H, 1:W + 1, :] = h1[j * H:(j + 1) * H]
    h1f = h1_ref[...].reshape(_B * F, Co)

    pieces2 = []
    for j in range(_B):
        hj = h1f[j * F:(j + 1) * F]
        pieces2.append(jnp.concatenate(
            [hj[0:R], hj[Wp:Wp + R], hj[2 * Wp:2 * Wp + R]], axis=1))
    xcat2 = jnp.concatenate(pieces2, axis=0)
    p2 = jnp.dot(xcat2, w2_ref[...], preferred_element_type=jnp.float32)
    p2 = p2.reshape(_B * H, Wp, 3 * Co)
    acc2 = (p2[:, 0:W, 0:Co] + p2[:, 1:W + 1, Co:2 * Co]
            + p2[:, 2:W + 2, 2 * Co:3 * Co])

    s2 = sb_ref[2:3].reshape(1, 1, Co)
    b2 = sb_ref[3:4].reshape(1, 1, Co)
    ssc = sb_ref[4:5].reshape(1, 1, Co)
    bsc = sb_ref[5:6].reshape(1, 1, Co)
    out = acc2 * s2 + b2 + (sc * ssc + bsc)
    out = jnp.maximum(out, 0.0)
    o_ref[...] = out.reshape(_B, H, W, Co)


def kernel(x, w1, g1, b1, m1, v1, w2, g2, b2, m2, v2,
           w_sc, g_sc, b_sc, m_sc, v_sc):
    N, Cin, H, W = x.shape
    Co = w1.shape[0]
    Wp = 64
    Hp = H + 2

    xf = jnp.transpose(x, (0, 2, 3, 1)).astype(jnp.bfloat16)
    xf = xf.reshape(N, H * W, Cin)

    s1, bb1 = _fold_bn(g1, b1, m1, v1)
    s2, bb2 = _fold_bn(g2, b2, m2, v2)
    ssc, bbsc = _fold_bn(g_sc, b_sc, m_sc, v_sc)
    z = jnp.zeros_like(s1)
    sb = jnp.stack([s1, bb1, s2, bb2, ssc, bbsc, z, z])

    wt1 = jnp.transpose(w1, (2, 1, 3, 0)).reshape(3 * Cin, 3 * Co)
    scb = jnp.zeros((3 * Cin, Co), jnp.float32)
    scb = scb.at[Cin:2 * Cin].set(w_sc[:, :, 0, 0].T)
    w1cat = jnp.concatenate([wt1, scb], axis=1).astype(jnp.bfloat16)
    w2cat = jnp.transpose(w2, (2, 1, 3, 0)).reshape(3 * Co, 3 * Co)
    w2cat = w2cat.astype(jnp.bfloat16)

    body = functools.partial(_block_body, H=H, W=W, Wp=Wp, Co=Co)
    out = pl.pallas_call(
        body,
        out_shape=jax.ShapeDtypeStruct((N, H, W, Co), jnp.float32),
        grid=(N // _B,),
        in_specs=[
            pl.BlockSpec((_B, H * W, Cin), lambda i: (i, 0, 0)),
            pl.BlockSpec((3 * Cin, 4 * Co), lambda i: (0, 0)),
            pl.BlockSpec((3 * Co, 3 * Co), lambda i: (0, 0)),
            pl.BlockSpec((8, Co), lambda i: (0, 0)),
        ],
        out_specs=pl.BlockSpec((_B, H, W, Co), lambda i: (i, 0, 0, 0)),
        scratch_shapes=[pltpu.VMEM((_B * Hp, Wp, Co), jnp.bfloat16),
                        pltpu.VMEM((_B * Hp, Wp, Cin), jnp.bfloat16)],
        compiler_params=pltpu.CompilerParams(
            dimension_semantics=("parallel",)),
    )(xf, w1cat, w2cat, sb)
    return jnp.transpose(out, (0, 3, 1, 2))


# speedup vs baseline: 1.0843x; 1.0010x over previous
import functools

import jax
import jax.numpy as jnp
from jax.experimental import pallas as pl
from jax.experimental.pallas import tpu as pltpu

_EPS = 1e-5
_B = 2


def _fold_bn(g, b, m, v):
    s = g / jnp.sqrt(v + _EPS)
    return s, b - m * s


def _block_body(x_ref, w1_ref, w2_ref, sb_ref, o_ref, h1_ref, xp_ref, *,
                H, W, Wp, Co):
    R = H * Wp
    F = (H + 2) * Wp
    Cin = x_ref.shape[2]
    for j in range(_B):
        b0 = j * (H + 2)
        xp_ref[b0:b0 + 1] = jnp.zeros((1, Wp, Cin), jnp.bfloat16)
        xp_ref[b0 + H + 1:b0 + H + 2] = jnp.zeros((1, Wp, Cin), jnp.bfloat16)
        xp_ref[b0 + 1:b0 + 1 + H, 0:1, :] = jnp.zeros((H, 1, Cin),
                                                      jnp.bfloat16)
        xp_ref[b0 + 1:b0 + 1 + H, W + 1:Wp, :] = jnp.zeros(
            (H, Wp - W - 1, Cin), jnp.bfloat16)
        xp_ref[b0 + 1:b0 + 1 + H, 1:W + 1, :] = x_ref[j].reshape(H, W, Cin)
    xflat = xp_ref[...].reshape(_B * F, Cin)
    pieces = []
    for j in range(_B):
        xm = xflat[j * F:(j + 1) * F]
        pieces.append(jnp.concatenate(
            [xm[0:R], xm[Wp:Wp + R], xm[2 * Wp:2 * Wp + R]], axis=1))
    xcat = jnp.concatenate(pieces, axis=0)
    p1a = jnp.dot(xcat, w1_ref[:, 0:2 * Co],
                  preferred_element_type=jnp.float32)
    p1a = p1a.reshape(_B * H, Wp, 2 * Co)
    acc1 = p1a[:, 0:W, 0:Co] + p1a[:, 1:W + 1, Co:2 * Co]
    p1b = jnp.dot(xcat, w1_ref[:, 2 * Co:4 * Co],
                  preferred_element_type=jnp.float32)
    p1b = p1b.reshape(_B * H, Wp, 2 * Co)
    acc1 = acc1 + p1b[:, 2:W + 2, 0:Co]
    sc = p1b[:, 1:W + 1, Co:2 * Co]

    s1 = sb_ref[0:1].reshape(1, 1, Co)
    b1 = sb_ref[1:2].reshape(1, 1, Co)
    h1 = jnp.maximum(acc1 * s1 + b1, 0.0).astype(jnp.bfloat16)

    for j in range(_B):
        b0 = j * (H + 2)
        h1_ref[b0:b0 + 1] = jnp.zeros((1, Wp, Co), jnp.bfloat16)
        h1_ref[b0 + H + 1:b0 + H + 2] = jnp.zeros((1, Wp, Co), jnp.bfloat16)
        h1_ref[b0 + 1:b0 + 1 + H, 0:1, :] = jnp.zeros((H, 1, Co),
                                                      jnp.bfloat16)
        h1_ref[b0 + 1:b0 + 1 + H, W + 1:Wp, :] = jnp.zeros(
            (H, Wp - W - 1, Co), jnp.bfloat16)
        h1_ref[b0 + 1:b0 + 1 + H, 1:W + 1, :] = h1[j * H:(j + 1) * H]
    h1f = h1_ref[...].reshape(_B * F, Co)

    pieces2 = []
    for j in range(_B):
        hj = h1f[j * F:(j + 1) * F]
        pieces2.append(jnp.concatenate(
            [hj[0:R], hj[Wp:Wp + R], hj[2 * Wp:2 * Wp + R]], axis=1))
    xcat2 = jnp.concatenate(pieces2, axis=0)
    p2a = jnp.dot(xcat2, w2_ref[:, 0:2 * Co],
                  preferred_element_type=jnp.float32)
    p2a = p2a.reshape(_B * H, Wp, 2 * Co)
    acc2 = p2a[:, 0:W, 0:Co] + p2a[:, 1:W + 1, Co:2 * Co]
    p2b = jnp.dot(xcat2, w2_ref[:, 2 * Co:3 * Co],
                  preferred_element_type=jnp.float32)
    p2b = p2b.reshape(_B * H, Wp, Co)
    acc2 = acc2 + p2b[:, 2:W + 2, :]

    s2 = sb_ref[2:3].reshape(1, 1, Co)
    b2 = sb_ref[3:4].reshape(1, 1, Co)
    ssc = sb_ref[4:5].reshape(1, 1, Co)
    bsc = sb_ref[5:6].reshape(1, 1, Co)
    out = acc2 * s2 + b2 + (sc * ssc + bsc)
    out = jnp.maximum(out, 0.0)
    o_ref[...] = out.reshape(_B, H, W, Co)


def kernel(x, w1, g1, b1, m1, v1, w2, g2, b2, m2, v2,
           w_sc, g_sc, b_sc, m_sc, v_sc):
    N, Cin, H, W = x.shape
    Co = w1.shape[0]
    Wp = 64
    Hp = H + 2

    xf = jnp.transpose(x, (0, 2, 3, 1)).astype(jnp.bfloat16)
    xf = xf.reshape(N, H * W, Cin)

    s1, bb1 = _fold_bn(g1, b1, m1, v1)
    s2, bb2 = _fold_bn(g2, b2, m2, v2)
    ssc, bbsc = _fold_bn(g_sc, b_sc, m_sc, v_sc)
    z = jnp.zeros_like(s1)
    sb = jnp.stack([s1, bb1, s2, bb2, ssc, bbsc, z, z])

    wt1 = jnp.transpose(w1, (2, 1, 3, 0)).reshape(3 * Cin, 3 * Co)
    scb = jnp.zeros((3 * Cin, Co), jnp.float32)
    scb = scb.at[Cin:2 * Cin].set(w_sc[:, :, 0, 0].T)
    w1cat = jnp.concatenate([wt1, scb], axis=1).astype(jnp.bfloat16)
    w2cat = jnp.transpose(w2, (2, 1, 3, 0)).reshape(3 * Co, 3 * Co)
    w2cat = w2cat.astype(jnp.bfloat16)

    body = functools.partial(_block_body, H=H, W=W, Wp=Wp, Co=Co)
    out = pl.pallas_call(
        body,
        out_shape=jax.ShapeDtypeStruct((N, H, W, Co), jnp.float32),
        grid=(N // _B,),
        in_specs=[
            pl.BlockSpec((_B, H * W, Cin), lambda i: (i, 0, 0)),
            pl.BlockSpec((3 * Cin, 4 * Co), lambda i: (0, 0)),
            pl.BlockSpec((3 * Co, 3 * Co), lambda i: (0, 0)),
            pl.BlockSpec((8, Co), lambda i: (0, 0)),
        ],
        out_specs=pl.BlockSpec((_B, H, W, Co), lambda i: (i, 0, 0, 0)),
        scratch_shapes=[pltpu.VMEM((_B * Hp, Wp, Co), jnp.bfloat16),
                        pltpu.VMEM((_B * Hp, Wp, Cin), jnp.bfloat16)],
        compiler_params=pltpu.CompilerParams(
            dimension_semantics=("parallel",)),
    )(xf, w1cat, w2cat, sb)
    return jnp.transpose(out, (0, 3, 1, 2))
